# baseline (speedup 1.0000x reference)
import functools

import jax
import jax.numpy as jnp
from jax import lax
from jax.experimental import pallas as pl
from jax.experimental.pallas import tpu as pltpu
from jax.experimental.pallas import tpu_sc as plsc


def _make_gather(V, D, B):
    info = plsc.get_sparse_core_info()
    NC, NS, L = info.num_cores, info.num_subcores, info.num_lanes
    NW = NC * NS
    b_per_w = B // NW
    CH = 128
    n_chunks = b_per_w // CH
    W = 2 * D
    mesh = plsc.VectorSubcoreMesh(core_axis_name="c", subcore_axis_name="s")

    @functools.partial(
        pl.kernel,
        mesh=mesh,
        out_type=jax.ShapeDtypeStruct((B, D), jnp.float32),
        scratch_types=[
            pltpu.VMEM((b_per_w,), jnp.int32),
            pltpu.VMEM((b_per_w,), jnp.int32),
            pltpu.VMEM((2, CH, W), jnp.float32),
            pltpu.VMEM((b_per_w, D), jnp.float32),
            pltpu.SemaphoreType.DMA,
            pltpu.SemaphoreType.DMA,
            pltpu.SemaphoreType.DMA,
        ],
    )
    def gather_kernel(idx_hbm, dense_hbm, out_hbm, idx_v, tidx_v, blocks_v,
                      stage_v, sem_in, sem_g0, sem_g1):
        wid = lax.axis_index("s") * NC + lax.axis_index("c")
        base = wid * b_per_w
        g_sems = (sem_g0, sem_g1)
        pltpu.async_copy(idx_hbm.at[pl.ds(base, b_per_w)], idx_v, sem_in).wait()

        def tidx_body(i, _):
            tidx_v[pl.ds(i * L, L)] = jax.lax.shift_right_logical(
                idx_v[pl.ds(i * L, L)], 1)
            return _

        lax.fori_loop(0, b_per_w // L, tidx_body, None)

        def fire(k):
            pltpu.async_copy(
                dense_hbm.at[tidx_v.at[pl.ds(k * CH, CH)]],
                blocks_v.at[k % 2], g_sems[k % 2])

        def wait_g(k):
            pltpu.make_async_copy(
                dense_hbm.at[pl.ds(0, CH)], blocks_v.at[k % 2],
                g_sems[k % 2]).wait()

        def extract(k):
            for i in range(CH // L):
                sub = (idx_v[pl.ds(k * CH + i * L, L)] & 1) * D
                for l in range(L):
                    h = sub[l]
                    for q in range(D // L):
                        stage_v[k * CH + i * L + l, pl.ds(q * L, L)] = (
                            blocks_v[k % 2, i * L + l, pl.ds(h + q * L, L)])

        fire(0)
        fire(1)
        for k in range(n_chunks):
            wait_g(k)
            extract(k)
            if k + 2 < n_chunks:
                fire(k + 2)
        pltpu.async_copy(stage_v, out_hbm.at[pl.ds(base, b_per_w)], sem_in).wait()

    return gather_kernel


def kernel(userid, table):
    B = userid.shape[0]
    V, D = table.shape
    dense = table.reshape(V // 2, 2 * D)
    gathered = _make_gather(V, D, B)(userid.astype(jnp.int32), dense)
    return gathered[:, None, :]


# speedup vs baseline: 1.3845x; 1.3845x over previous
import functools

import jax
import jax.numpy as jnp
from jax import lax
from jax.experimental import pallas as pl
from jax.experimental.pallas import tpu as pltpu
from jax.experimental.pallas import tpu_sc as plsc


def _make_gather(V, D, B):
    info = plsc.get_sparse_core_info()
    NC, NS, L = info.num_cores, info.num_subcores, info.num_lanes
    NW = NC * NS
    b_per_w = B // NW
    mesh = plsc.VectorSubcoreMesh(core_axis_name="c", subcore_axis_name="s")

    @functools.partial(
        pl.kernel,
        mesh=mesh,
        out_type=jax.ShapeDtypeStruct((B, D), jnp.float32),
        scratch_types=[
            pltpu.VMEM((b_per_w,), jnp.int32),
            pltpu.VMEM((320, D), jnp.float32),
            pltpu.SemaphoreType.DMA,
            pltpu.SemaphoreType.DMA,
            pltpu.SemaphoreType.DMA,
        ],
    )
    def gather_kernel(idx_hbm, table_hbm, out_hbm, idx_v, rows_v, sem_in,
                      sem_row, sem_out):
        wid = lax.axis_index("s") * NC + lax.axis_index("c")
        base = wid * b_per_w
        nv = 320
        pltpu.async_copy(idx_hbm.at[pl.ds(base, b_per_w)], idx_v, sem_in).wait()

        def fire_vmem(c, _):
            vec = idx_v[pl.ds(c * L, L)]
            for l in range(L):
                pltpu.async_copy(
                    table_hbm.at[vec[l]], rows_v.at[c * L + l], sem_row)
            return _

        def fire_hbm(c, _):
            vec = idx_v[pl.ds(c * L, L)]
            for l in range(L):
                pltpu.async_copy(
                    table_hbm.at[vec[l]], out_hbm.at[base + c * L + l],
                    sem_out)
            return _

        lax.fori_loop(0, nv // L, fire_vmem, None)
        lax.fori_loop(nv // L, b_per_w // L, fire_hbm, None)
        pltpu.make_async_copy(
            table_hbm.at[pl.ds(0, b_per_w - nv)],
            out_hbm.at[pl.ds(base + nv, b_per_w - nv)], sem_out).wait()
        pltpu.make_async_copy(
            table_hbm.at[pl.ds(0, nv)], rows_v, sem_row).wait()
        pltpu.async_copy(rows_v, out_hbm.at[pl.ds(base, nv)], sem_in).wait()

    return gather_kernel


def kernel(userid, table):
    B = userid.shape[0]
    V, D = table.shape
    gathered = _make_gather(V, D, B)(userid.astype(jnp.int32), table)
    return gathered[:, None, :]


# speedup vs baseline: 1.7325x; 1.2514x over previous
import functools

import jax
import jax.numpy as jnp
from jax import lax
from jax.experimental import pallas as pl
from jax.experimental.pallas import tpu as pltpu
from jax.experimental.pallas import tpu_sc as plsc


def _make_gather(V, D, B):
    info = plsc.get_sparse_core_info()
    NC, NS, L = info.num_cores, info.num_subcores, info.num_lanes
    NW = NC * NS
    b_per_w = B // NW
    mesh = plsc.VectorSubcoreMesh(core_axis_name="c", subcore_axis_name="s")

    @functools.partial(
        pl.kernel,
        mesh=mesh,
        out_type=jax.ShapeDtypeStruct((B, D), jnp.float32),
        scratch_types=[
            pltpu.VMEM((b_per_w,), jnp.int32),
            pltpu.VMEM((b_per_w, D), jnp.float32),
            pltpu.SemaphoreType.DMA,
            pltpu.SemaphoreType.DMA,
        ],
    )
    def gather_kernel(idx_hbm, table_hbm, out_hbm, idx_v, rows_v, sem_in,
                      sem_row):
        wid = lax.axis_index("s") * NC + lax.axis_index("c")
        base = wid * b_per_w
        pltpu.async_copy(idx_hbm.at[pl.ds(base, b_per_w)], idx_v, sem_in).wait()

        def fire_chunk(c, _):
            vec = idx_v[pl.ds(c * L, L)]
            for l in range(L):
                pltpu.async_copy(
                    table_hbm.at[vec[l]], rows_v.at[c * L + l], sem_row)
            return _

        lax.fori_loop(0, b_per_w // L, fire_chunk, None)
        pltpu.make_async_copy(
            table_hbm.at[pl.ds(0, b_per_w)], rows_v, sem_row).wait()
        pltpu.async_copy(rows_v, out_hbm.at[pl.ds(base, b_per_w)], sem_in).wait()

    return gather_kernel


def kernel(userid, table):
    B = userid.shape[0]
    V, D = table.shape
    gathered = _make_gather(V, D, B)(userid.astype(jnp.int32), table)
    return gathered[:, None, :]
